# baseline (speedup 1.0000x reference)
import functools

import jax
import jax.numpy as jnp
from jax import lax
from jax.experimental import pallas as pl
from jax.experimental.pallas import tpu as pltpu
from jax.experimental.pallas import tpu_sc as plsc

G = 2
K = 8192
D = 256
M = 2048
KBLK = 1024
NKB = K // KBLK


def _layer_step(xs_ref, xd_ref, cb_ref, kb, idx_ref, accv, acckb, xnb, rbuf):
    @pl.when(kb == 0)
    def _():
        if xd_ref is None:
            r = xs_ref[0]
        else:
            r = xs_ref[0] + 2.0 * xd_ref[0]
        rbuf[...] = r
        xn = jnp.sum(r * r, axis=1) * 0.25
        xnb[...] = jnp.broadcast_to(xn[:, None], (M, KBLK))

    cb = cb_ref[0]
    cn = jnp.sum(cb * cb, axis=1)
    d = lax.dot_general(rbuf[...], cb, (((1,), (1,)), ((), ())),
                        preferred_element_type=jnp.float32)
    s = (xnb[...] + d) + cn[None, :]

    @pl.when(kb == 0)
    def _():
        accv[...] = s
        acckb[...] = jnp.zeros((M, KBLK), jnp.int8)

    @pl.when(kb > 0)
    def _():
        prev = accv[...]
        upd = s < prev
        accv[...] = jnp.minimum(s, prev)
        acckb[...] = jnp.where(upd, jnp.full((M, KBLK), kb, jnp.int8),
                               acckb[...])

    @pl.when(kb == NKB - 1)
    def _():
        av = accv[...]
        m = jnp.min(av, axis=1)
        jcol = lax.broadcasted_iota(jnp.int32, (M, KBLK), 1)
        gidx = acckb[...].astype(jnp.int32) * KBLK + jcol
        cand = jnp.where(av == m[:, None], gidx, K)
        idx_ref[0] = jnp.min(cand, axis=1)[None]


def _layer1_body(x_ref, cb_ref, idx_ref, accv, acckb, xnb, rbuf):
    _layer_step(x_ref, None, cb_ref, pl.program_id(1), idx_ref, accv, acckb,
                xnb, rbuf)


def _layer2_body(x_ref, xd_ref, cb_ref, idx_ref, accv, acckb, xnb, rbuf):
    _layer_step(x_ref, xd_ref, cb_ref, pl.program_id(1), idx_ref, accv,
                acckb, xnb, rbuf)


def _tc_gather_body(idx_ref, cb_ref, o_ref, cnt_ref, idxb):
    kb = pl.program_id(1)

    @pl.when(kb == 0)
    def _():
        idxb[...] = jnp.broadcast_to(idx_ref[0], (M, KBLK))

    oh = (idxb[...] == kb * KBLK +
          lax.broadcasted_iota(jnp.int32, (1, KBLK), 1)).astype(jnp.float32)
    cb = cb_ref[0]
    mask = jnp.uint32(0xFFFF0000)
    hi = lax.bitcast_convert_type(
        lax.bitcast_convert_type(cb, jnp.uint32) & mask, jnp.float32)
    mf = cb - hi
    mid = lax.bitcast_convert_type(
        lax.bitcast_convert_type(mf, jnp.uint32) & mask, jnp.float32)
    lo = mf - mid
    dims = (((1,), (0,)), ((), ()))
    d = (lax.dot_general(oh, hi, dims, preferred_element_type=jnp.float32)
         + lax.dot_general(oh, mid, dims, preferred_element_type=jnp.float32)
         ) + lax.dot_general(oh, lo, dims, preferred_element_type=jnp.float32)

    @pl.when(kb == 0)
    def _():
        o_ref[0] = d

    @pl.when(kb > 0)
    def _():
        o_ref[0] = o_ref[0] + d

    if cnt_ref is not None:
        cnt_ref[0] = jnp.sum(oh, axis=0, keepdims=True)


def _run_tc_gather(idxcol, cb, with_counts=False):
    outs = [jax.ShapeDtypeStruct((G, M, D), jnp.float32)]
    out_specs = [_x_spec]
    body = _tc_gather_body
    if with_counts:
        outs.append(jax.ShapeDtypeStruct((G, 1, K), jnp.float32))
        out_specs.append(pl.BlockSpec((1, 1, KBLK), lambda g, kb: (g, 0, kb)))
    else:
        def body(idx_ref, cb_ref, o_ref, idxb):
            return _tc_gather_body(idx_ref, cb_ref, o_ref, None, idxb)
    r = pl.pallas_call(
        body,
        grid=(G, NKB),
        in_specs=[pl.BlockSpec((1, M, 1), lambda g, kb: (g, 0, 0)),
                  _cb_spec],
        out_specs=out_specs,
        out_shape=outs,
        scratch_shapes=[pltpu.VMEM((M, KBLK), jnp.int32)],
        compiler_params=pltpu.CompilerParams(
            vmem_limit_bytes=100 * 1024 * 1024),
    )(idxcol, cb)
    return r if with_counts else r[0]


def _finalize_body(x_ref, xd0_ref, xd1_ref, cnt_ref, q_ref, cp_ref, perp_ref):
    xf = x_ref[0]
    a = xd0_ref[0]
    b = xd1_ref[0]
    q_ref[0] = a + b
    r0 = xf - a
    c0 = jnp.sum((r0 * r0).reshape(1, M * D), axis=1, keepdims=True)
    e = r0 - b
    c1 = jnp.sum((e * e).reshape(1, M * D), axis=1, keepdims=True)
    cp_ref[0] = (c0 + c1) * (1.0 / (M * D))
    prob = cnt_ref[0] * (1.0 / M)
    h = jnp.sum(prob * jnp.log(prob + 1e-07), axis=1, keepdims=True)
    perp_ref[0] = jnp.exp(-h)


_x_spec = pl.BlockSpec((1, M, D), lambda g, kb: (g, 0, 0))
_cb_spec = pl.BlockSpec((1, KBLK, D), lambda g, kb: (g, kb, 0))
_idx_spec = pl.BlockSpec((1, 1, M), lambda g, kb: (g, 0, 0))


def _run_layer1(xf, cb):
    return pl.pallas_call(
        _layer1_body,
        grid=(G, NKB),
        in_specs=[_x_spec, _cb_spec],
        out_specs=[_idx_spec],
        out_shape=[jax.ShapeDtypeStruct((G, 1, M), jnp.int32)],
        scratch_shapes=[pltpu.VMEM((M, KBLK), jnp.float32),
                        pltpu.VMEM((M, KBLK), jnp.int8),
                        pltpu.VMEM((M, KBLK), jnp.float32),
                        pltpu.VMEM((M, D), jnp.float32)],
        compiler_params=pltpu.CompilerParams(
            vmem_limit_bytes=100 * 1024 * 1024),
    )(xf, cb)[0]


def _run_layer2(xf, xd0, cb):
    return pl.pallas_call(
        _layer2_body,
        grid=(G, NKB),
        in_specs=[_x_spec, _x_spec, _cb_spec],
        out_specs=[_idx_spec],
        out_shape=[jax.ShapeDtypeStruct((G, 1, M), jnp.int32)],
        scratch_shapes=[pltpu.VMEM((M, KBLK), jnp.float32),
                        pltpu.VMEM((M, KBLK), jnp.int8),
                        pltpu.VMEM((M, KBLK), jnp.float32),
                        pltpu.VMEM((M, D), jnp.float32)],
        compiler_params=pltpu.CompilerParams(
            vmem_limit_bytes=100 * 1024 * 1024),
    )(xf, xd0, cb)[0]


def _run_finalize(xf, xd0, xd1, counts):
    s_x = pl.BlockSpec((1, M, D), lambda g: (g, 0, 0))
    s_s = pl.BlockSpec((1, 1, 1), lambda g: (g, 0, 0))
    s_c = pl.BlockSpec((1, 1, K), lambda g: (g, 0, 0))
    return pl.pallas_call(
        _finalize_body,
        grid=(G,),
        in_specs=[s_x, s_x, s_x, s_c],
        out_specs=[s_x, s_s, s_s],
        out_shape=[jax.ShapeDtypeStruct((G, M, D), jnp.float32),
                   jax.ShapeDtypeStruct((G, 1, 1), jnp.float32),
                   jax.ShapeDtypeStruct((G, 1, 1), jnp.float32)],
    )(xf, xd0, xd1, counts)


_NC, _NS = 2, 16
_RPW = (G * M) // (_NC * _NS)
_CPW = K // _NS


def _sc_counts(lidx_flat, ones, zeros):
    mesh = plsc.VectorSubcoreMesh(core_axis_name="c", subcore_axis_name="s")

    @functools.partial(
        pl.kernel, mesh=mesh,
        out_type=[jax.ShapeDtypeStruct((G * K,), jnp.float32)],
        scratch_types=[pltpu.VMEM((_RPW,), jnp.int32),
                       pltpu.VMEM((_RPW,), jnp.float32),
                       pltpu.VMEM_SHARED((K,), jnp.float32)])
    def k(lidx_hbm, ones_hbm, zeros_hbm, cnt_hbm, lidx_v, ones_v, shared):
        c = lax.axis_index("c")
        s = lax.axis_index("s")
        base = c * M + s * _RPW
        pltpu.sync_copy(lidx_hbm.at[pl.ds(base, _RPW)], lidx_v)
        pltpu.sync_copy(ones_hbm.at[pl.ds(base, _RPW)], ones_v)
        pltpu.sync_copy(zeros_hbm.at[pl.ds(s * _CPW, _CPW)],
                        shared.at[pl.ds(s * _CPW, _CPW)])
        plsc.subcore_barrier()
        pltpu.sync_copy(ones_v, shared.at[lidx_v], add=True)
        plsc.subcore_barrier()
        pltpu.sync_copy(shared.at[pl.ds(s * _CPW, _CPW)],
                        cnt_hbm.at[pl.ds(c * K + s * _CPW, _CPW)])

    return k(lidx_flat, ones, zeros)[0]


def kernel(x, codebooks):
    xt = jnp.transpose(x, (0, 2, 1)).reshape(M, G * D)
    xf = jnp.stack([xt[:, g * D:(g + 1) * D] for g in range(G)])
    cb0 = codebooks[:, 0]
    cb1 = codebooks[:, 1]

    ones = jnp.ones((G * M,), jnp.float32)
    zeros = jnp.zeros((K,), jnp.float32)

    xs = -2.0 * xf
    idx0 = _run_layer1(xs, cb0)
    xd0, counts = _run_tc_gather(idx0.reshape(G, M, 1), cb0,
                                 with_counts=True)
    idx1 = _run_layer2(xs, xd0, cb1)
    xd1 = _run_tc_gather(idx1.reshape(G, M, 1), cb1)
    q, cp, perp = _run_finalize(xf, xd0, xd1, counts)

    quantized = jnp.transpose(
        jnp.concatenate([q[g].reshape(4, 512, D) for g in range(G)], axis=2),
        (0, 2, 1))
    commit_total = cp[0, 0, 0] + cp[1, 0, 0]
    return quantized, commit_total, perp.reshape(G), idx0.reshape(G, M)


# speedup vs baseline: 1.3425x; 1.3425x over previous
import functools

import jax
import jax.numpy as jnp
from jax import lax
from jax.experimental import pallas as pl
from jax.experimental.pallas import tpu as pltpu
from jax.experimental.pallas import tpu_sc as plsc

G = 2
NUMQ = 2
K = 8192
D = 256
M = 2048
KBLK = 1024
NKB = K // KBLK


def _layer_step(xs_ref, xd_ref, cb_ref, kb, idx_ref, accv, acckb, xnb, rbuf):
    @pl.when(kb == 0)
    def _():
        if xd_ref is None:
            r = xs_ref[0]
        else:
            r = xs_ref[0] + 2.0 * xd_ref[0]
        rbuf[...] = r
        xn = jnp.sum(r * r, axis=1) * 0.25
        xnb[...] = jnp.broadcast_to(xn[:, None], (M, KBLK))

    cb = cb_ref[0]
    cn = jnp.sum(cb * cb, axis=1)
    d = lax.dot_general(rbuf[...], cb, (((1,), (1,)), ((), ())),
                        preferred_element_type=jnp.float32)
    s = (xnb[...] + d) + cn[None, :]

    @pl.when(kb == 0)
    def _():
        accv[...] = s
        acckb[...] = jnp.zeros((M, KBLK), jnp.int8)

    @pl.when(kb > 0)
    def _():
        prev = accv[...]
        upd = s < prev
        accv[...] = jnp.minimum(s, prev)
        acckb[...] = jnp.where(upd, jnp.full((M, KBLK), kb, jnp.int8),
                               acckb[...])

    @pl.when(kb == NKB - 1)
    def _():
        av = accv[...]
        m = jnp.min(av, axis=1)
        jcol = lax.broadcasted_iota(jnp.int32, (M, KBLK), 1)
        gidx = acckb[...].astype(jnp.int32) * KBLK + jcol
        cand = jnp.where(av == m[:, None], gidx, K)
        idx_ref[0] = jnp.min(cand, axis=1)[None]


def _layer1_body(x_ref, cb_ref, idx_ref, accv, acckb, xnb, rbuf):
    _layer_step(x_ref, None, cb_ref, pl.program_id(1), idx_ref, accv, acckb,
                xnb, rbuf)


def _layer2_body(x_ref, xd_ref, cb_ref, idx_ref, accv, acckb, xnb, rbuf):
    _layer_step(x_ref, xd_ref, cb_ref, pl.program_id(1), idx_ref, accv,
                acckb, xnb, rbuf)


def _finalize_body(x_ref, xd0_ref, xd1_ref, cnt_ref, q_ref, cp_ref, perp_ref):
    xf = x_ref[0]
    a = xd0_ref[0]
    b = xd1_ref[0]
    q_ref[0] = a + b
    r0 = xf - a
    c0 = jnp.sum((r0 * r0).reshape(1, M * D), axis=1, keepdims=True)
    e = r0 - b
    c1 = jnp.sum((e * e).reshape(1, M * D), axis=1, keepdims=True)
    cp_ref[0] = (c0 + c1) * (1.0 / (M * D))
    counts = cnt_ref[0, 0:1, :] + cnt_ref[0, 1:2, :]
    prob = counts * (1.0 / M)
    h = jnp.sum(prob * jnp.log(prob + 1e-07), axis=1, keepdims=True)
    perp_ref[0] = jnp.exp(-h)


_x_spec = pl.BlockSpec((1, M, D), lambda g, kb: (g, 0, 0))
_cb_spec = pl.BlockSpec((1, KBLK, D), lambda g, kb: (g, kb, 0))
_idx_spec = pl.BlockSpec((1, 1, M), lambda g, kb: (g, 0, 0))


def _run_layer1(xf, cb):
    gg = xf.shape[0]
    return pl.pallas_call(
        _layer1_body,
        grid=(gg, NKB),
        in_specs=[_x_spec, _cb_spec],
        out_specs=[_idx_spec],
        out_shape=[jax.ShapeDtypeStruct((gg, 1, M), jnp.int32)],
        scratch_shapes=[pltpu.VMEM((M, KBLK), jnp.float32),
                        pltpu.VMEM((M, KBLK), jnp.int8),
                        pltpu.VMEM((M, KBLK), jnp.float32),
                        pltpu.VMEM((M, D), jnp.float32)],
        compiler_params=pltpu.CompilerParams(
            vmem_limit_bytes=100 * 1024 * 1024),
    )(xf, cb)[0]


def _run_layer2(xf, xd0, cb):
    gg = xf.shape[0]
    return pl.pallas_call(
        _layer2_body,
        grid=(gg, NKB),
        in_specs=[_x_spec, _x_spec, _cb_spec],
        out_specs=[_idx_spec],
        out_shape=[jax.ShapeDtypeStruct((gg, 1, M), jnp.int32)],
        scratch_shapes=[pltpu.VMEM((M, KBLK), jnp.float32),
                        pltpu.VMEM((M, KBLK), jnp.int8),
                        pltpu.VMEM((M, KBLK), jnp.float32),
                        pltpu.VMEM((M, D), jnp.float32)],
        compiler_params=pltpu.CompilerParams(
            vmem_limit_bytes=100 * 1024 * 1024),
    )(xf, xd0, cb)[0]


def _run_finalize(xf, xd0, xd1, counts):
    s_x = pl.BlockSpec((1, M, D), lambda g: (g, 0, 0))
    s_s = pl.BlockSpec((1, 1, 1), lambda g: (g, 0, 0))
    s_c = pl.BlockSpec((1, _NC, K), lambda g: (g, 0, 0))
    return pl.pallas_call(
        _finalize_body,
        grid=(G,),
        in_specs=[s_x, s_x, s_x, s_c],
        out_specs=[s_x, s_s, s_s],
        out_shape=[jax.ShapeDtypeStruct((G, M, D), jnp.float32),
                   jax.ShapeDtypeStruct((G, 1, 1), jnp.float32),
                   jax.ShapeDtypeStruct((G, 1, 1), jnp.float32)],
    )(xf, xd0, xd1, counts)


_NC, _NS = 2, 16
_RPW = M // (_NC * _NS)
_CPW = K // _NS


def _sc_gather(cb_flat, gidx_flat, lidx_flat=None, ones=None, zeros=None):
    mesh = plsc.VectorSubcoreMesh(core_axis_name="c", subcore_axis_name="s")
    with_counts = lidx_flat is not None
    out_type = [jax.ShapeDtypeStruct((M, D), jnp.float32)]
    if with_counts:
        out_type.append(jax.ShapeDtypeStruct((_NC * K,), jnp.float32))
    scratch = [
        pltpu.VMEM((_RPW,), jnp.int32),
        pltpu.VMEM((_RPW, D), jnp.float32),
        pltpu.SemaphoreType.DMA,
    ]
    if with_counts:
        scratch += [pltpu.VMEM((_RPW,), jnp.int32),
                    pltpu.VMEM((_RPW,), jnp.float32),
                    pltpu.VMEM_SHARED((K,), jnp.float32)]

    @functools.partial(pl.kernel, mesh=mesh, out_type=out_type,
                       scratch_types=scratch)
    def k(cb_hbm, gidx_hbm, *rest):
        if with_counts:
            (lidx_hbm, ones_hbm, zeros_hbm, out_hbm, cnt_hbm,
             idx_v, rows_v, sem, lidx_v, ones_v, shared) = rest
        else:
            out_hbm, idx_v, rows_v, sem = rest
        c = lax.axis_index("c")
        s = lax.axis_index("s")
        base = (s * _NC + c) * _RPW
        pltpu.sync_copy(gidx_hbm.at[pl.ds(base, _RPW)], idx_v)

        if with_counts:
            pltpu.sync_copy(lidx_hbm.at[pl.ds(base, _RPW)], lidx_v)
            pltpu.sync_copy(ones_hbm.at[pl.ds(base, _RPW)], ones_v)
            pltpu.sync_copy(zeros_hbm.at[pl.ds(s * _CPW, _CPW)],
                            shared.at[pl.ds(s * _CPW, _CPW)])
            plsc.subcore_barrier()
            pltpu.sync_copy(ones_v, shared.at[lidx_v], add=True)
            plsc.subcore_barrier()
            pltpu.sync_copy(shared.at[pl.ds(s * _CPW, _CPW)],
                            cnt_hbm.at[pl.ds(c * K + s * _CPW, _CPW)])

        pltpu.async_copy(cb_hbm.at[idx_v], rows_v, sem).wait()
        pltpu.sync_copy(rows_v, out_hbm.at[pl.ds(base, _RPW)])

    if with_counts:
        return k(cb_flat, gidx_flat, lidx_flat, ones, zeros)
    return k(cb_flat, gidx_flat)


def kernel(x, codebooks):
    xt = jnp.transpose(x, (0, 2, 1)).reshape(M, G * D)
    xf = jnp.stack([xt[:, g * D:(g + 1) * D] for g in range(G)])
    cb0 = codebooks[:, 0]
    cb1 = codebooks[:, 1]

    ones = jnp.ones((M,), jnp.float32)
    zeros = jnp.zeros((K,), jnp.float32)
    cb_flat = codebooks.reshape(G * NUMQ * K, D)

    xs = -2.0 * xf
    idx0s, idx1s, xd0s, xd1s, cnts = [], [], [], [], []
    for g in range(G):
        idx0 = _run_layer1(xs[g:g + 1], cb0[g:g + 1])
        xd0, cnt = _sc_gather(cb_flat,
                              idx0.reshape(M) + (g * NUMQ) * K,
                              idx0.reshape(M), ones, zeros)
        idx1 = _run_layer2(xs[g:g + 1], xd0[None], cb1[g:g + 1])
        xd1 = _sc_gather(cb_flat,
                         idx1.reshape(M) + (g * NUMQ + 1) * K)[0]
        idx0s.append(idx0)
        idx1s.append(idx1)
        xd0s.append(xd0)
        xd1s.append(xd1)
        cnts.append(cnt.reshape(1, _NC, K))

    idx0 = jnp.concatenate(idx0s, axis=0)
    xd0 = jnp.stack(xd0s)
    xd1 = jnp.stack(xd1s)
    q, cp, perp = _run_finalize(xf, xd0, xd1, jnp.concatenate(cnts, axis=0))

    quantized = jnp.transpose(
        jnp.concatenate([q[g].reshape(4, 512, D) for g in range(G)], axis=2),
        (0, 2, 1))
    commit_total = cp[0, 0, 0] + cp[1, 0, 0]
    return quantized, commit_total, perp.reshape(G), idx0.reshape(G, M)


# speedup vs baseline: 1.4313x; 1.0661x over previous
import functools

import jax
import jax.numpy as jnp
from jax import lax
from jax.experimental import pallas as pl
from jax.experimental.pallas import tpu as pltpu
from jax.experimental.pallas import tpu_sc as plsc

G = 2
K = 8192
D = 256
M = 2048
KBLK = 1024
NKB = K // KBLK


def _layer_step(xs_ref, xd_ref, cb_ref, kb, idx_ref, accv, acckb, xnb, rbuf):
    @pl.when(kb == 0)
    def _():
        if xd_ref is None:
            r = xs_ref[0]
        else:
            r = xs_ref[0] + 2.0 * xd_ref[0]
        rbuf[...] = r
        xn = jnp.sum(r * r, axis=1) * 0.25
        xnb[...] = jnp.broadcast_to(xn[:, None], (M, KBLK))

    cb = cb_ref[0]
    cn = jnp.sum(cb * cb, axis=1)
    d = lax.dot_general(rbuf[...], cb, (((1,), (1,)), ((), ())),
                        preferred_element_type=jnp.float32)
    s = (xnb[...] + d) + cn[None, :]

    @pl.when(kb == 0)
    def _():
        accv[...] = s
        acckb[...] = jnp.zeros((M, KBLK), jnp.int8)

    @pl.when(kb > 0)
    def _():
        prev = accv[...]
        upd = s < prev
        accv[...] = jnp.minimum(s, prev)
        acckb[...] = jnp.where(upd, jnp.full((M, KBLK), kb, jnp.int8),
                               acckb[...])

    @pl.when(kb == NKB - 1)
    def _():
        av = accv[...]
        m = jnp.min(av, axis=1)
        jcol = lax.broadcasted_iota(jnp.int32, (M, KBLK), 1)
        gidx = acckb[...].astype(jnp.int32) * KBLK + jcol
        cand = jnp.where(av == m[:, None], gidx, K)
        idx_ref[0] = jnp.min(cand, axis=1)[None]


def _layer1_body(x_ref, cb_ref, idx_ref, accv, acckb, xnb, rbuf):
    _layer_step(x_ref, None, cb_ref, pl.program_id(1), idx_ref, accv, acckb,
                xnb, rbuf)


def _layer2_body(x_ref, xd_ref, cb_ref, idx_ref, accv, acckb, xnb, rbuf):
    _layer_step(x_ref, xd_ref, cb_ref, pl.program_id(1), idx_ref, accv,
                acckb, xnb, rbuf)


def _finalize_body(x_ref, xd0_ref, xd1_ref, cnt_ref, q_ref, cp_ref, perp_ref):
    xf = x_ref[0]
    a = xd0_ref[0]
    b = xd1_ref[0]
    q_ref[0] = a + b
    r0 = xf - a
    c0 = jnp.sum((r0 * r0).reshape(1, M * D), axis=1, keepdims=True)
    e = r0 - b
    c1 = jnp.sum((e * e).reshape(1, M * D), axis=1, keepdims=True)
    cp_ref[0] = (c0 + c1) * (1.0 / (M * D))
    prob = cnt_ref[0] * (1.0 / M)
    h = jnp.sum(prob * jnp.log(prob + 1e-07), axis=1, keepdims=True)
    perp_ref[0] = jnp.exp(-h)


_x_spec = pl.BlockSpec((1, M, D), lambda g, kb: (g, 0, 0))
_cb_spec = pl.BlockSpec((1, KBLK, D), lambda g, kb: (g, kb, 0))
_idx_spec = pl.BlockSpec((1, 1, M), lambda g, kb: (g, 0, 0))


def _run_layer1(xf, cb):
    return pl.pallas_call(
        _layer1_body,
        grid=(G, NKB),
        in_specs=[_x_spec, _cb_spec],
        out_specs=[_idx_spec],
        out_shape=[jax.ShapeDtypeStruct((G, 1, M), jnp.int32)],
        scratch_shapes=[pltpu.VMEM((M, KBLK), jnp.float32),
                        pltpu.VMEM((M, KBLK), jnp.int8),
                        pltpu.VMEM((M, KBLK), jnp.float32),
                        pltpu.VMEM((M, D), jnp.float32)],
        compiler_params=pltpu.CompilerParams(
            vmem_limit_bytes=100 * 1024 * 1024),
    )(xf, cb)[0]


def _run_layer2(xf, xd0, cb):
    return pl.pallas_call(
        _layer2_body,
        grid=(G, NKB),
        in_specs=[_x_spec, _x_spec, _cb_spec],
        out_specs=[_idx_spec],
        out_shape=[jax.ShapeDtypeStruct((G, 1, M), jnp.int32)],
        scratch_shapes=[pltpu.VMEM((M, KBLK), jnp.float32),
                        pltpu.VMEM((M, KBLK), jnp.int8),
                        pltpu.VMEM((M, KBLK), jnp.float32),
                        pltpu.VMEM((M, D), jnp.float32)],
        compiler_params=pltpu.CompilerParams(
            vmem_limit_bytes=100 * 1024 * 1024),
    )(xf, xd0, cb)[0]


def _run_finalize(xf, xd0, xd1, counts):
    s_x = pl.BlockSpec((1, M, D), lambda g: (g, 0, 0))
    s_s = pl.BlockSpec((1, 1, 1), lambda g: (g, 0, 0))
    s_c = pl.BlockSpec((1, 1, K), lambda g: (g, 0, 0))
    return pl.pallas_call(
        _finalize_body,
        grid=(G,),
        in_specs=[s_x, s_x, s_x, s_c],
        out_specs=[s_x, s_s, s_s],
        out_shape=[jax.ShapeDtypeStruct((G, M, D), jnp.float32),
                   jax.ShapeDtypeStruct((G, 1, 1), jnp.float32),
                   jax.ShapeDtypeStruct((G, 1, 1), jnp.float32)],
    )(xf, xd0, xd1, counts)


_NC, _NS = 2, 16
_RPW = (G * M) // (_NC * _NS)
_CPW = K // _NS


def _sc_gather(cb_flat, gidx_flat, lidx_flat=None, ones=None, zeros=None):
    mesh = plsc.VectorSubcoreMesh(core_axis_name="c", subcore_axis_name="s")
    with_counts = lidx_flat is not None
    out_type = [jax.ShapeDtypeStruct((G * M, D), jnp.float32)]
    if with_counts:
        out_type.append(jax.ShapeDtypeStruct((G * K,), jnp.float32))
    scratch = [
        pltpu.VMEM((_RPW,), jnp.int32),
        pltpu.VMEM((_RPW, D), jnp.float32),
        pltpu.SemaphoreType.DMA,
    ]
    if with_counts:
        scratch += [pltpu.VMEM((_RPW,), jnp.int32),
                    pltpu.VMEM((_RPW,), jnp.float32),
                    pltpu.VMEM_SHARED((K,), jnp.float32)]

    @functools.partial(pl.kernel, mesh=mesh, out_type=out_type,
                       scratch_types=scratch)
    def k(cb_hbm, gidx_hbm, *rest):
        if with_counts:
            (lidx_hbm, ones_hbm, zeros_hbm, out_hbm, cnt_hbm,
             idx_v, rows_v, sem, lidx_v, ones_v, shared) = rest
        else:
            out_hbm, idx_v, rows_v, sem = rest
        c = lax.axis_index("c")
        s = lax.axis_index("s")
        base = c * M + s * _RPW
        pltpu.sync_copy(gidx_hbm.at[pl.ds(base, _RPW)], idx_v)

        if with_counts:
            pltpu.sync_copy(lidx_hbm.at[pl.ds(base, _RPW)], lidx_v)
            pltpu.sync_copy(ones_hbm.at[pl.ds(base, _RPW)], ones_v)
            pltpu.sync_copy(zeros_hbm.at[pl.ds(s * _CPW, _CPW)],
                            shared.at[pl.ds(s * _CPW, _CPW)])
            plsc.subcore_barrier()
            pltpu.sync_copy(ones_v, shared.at[lidx_v], add=True)
            plsc.subcore_barrier()
            pltpu.sync_copy(shared.at[pl.ds(s * _CPW, _CPW)],
                            cnt_hbm.at[pl.ds(c * K + s * _CPW, _CPW)])

        pltpu.async_copy(cb_hbm.at[idx_v], rows_v, sem).wait()
        pltpu.sync_copy(rows_v, out_hbm.at[pl.ds(base, _RPW)])

    if with_counts:
        return k(cb_flat, gidx_flat, lidx_flat, ones, zeros)
    return k(cb_flat, gidx_flat)


def kernel(x, codebooks):
    xt = jnp.transpose(x, (0, 2, 1)).reshape(M, G * D)
    xf = jnp.stack([xt[:, g * D:(g + 1) * D] for g in range(G)])
    cb0 = codebooks[:, 0]
    cb1 = codebooks[:, 1]

    goff = (jnp.arange(G, dtype=jnp.int32) * K)[:, None]
    ones = jnp.ones((G * M,), jnp.float32)
    zeros = jnp.zeros((K,), jnp.float32)

    xs = -2.0 * xf
    idx0 = _run_layer1(xs, cb0)
    xd0, counts = _sc_gather(cb0.reshape(G * K, D),
                             (idx0.reshape(G, M) + goff).reshape(G * M),
                             idx0.reshape(G * M), ones, zeros)
    xd0 = xd0.reshape(G, M, D)
    idx1 = _run_layer2(xs, xd0, cb1)
    xd1 = _sc_gather(cb1.reshape(G * K, D),
                     (idx1.reshape(G, M) + goff).reshape(G * M)
                     )[0].reshape(G, M, D)
    q, cp, perp = _run_finalize(xf, xd0, xd1, counts.reshape(G, 1, K))

    quantized = jnp.transpose(
        jnp.concatenate([q[g].reshape(4, 512, D) for g in range(G)], axis=2),
        (0, 2, 1))
    commit_total = cp[0, 0, 0] + cp[1, 0, 0]
    return quantized, commit_total, perp.reshape(G), idx0.reshape(G, M)


# speedup vs baseline: 1.6598x; 1.1596x over previous
import functools

import jax
import jax.numpy as jnp
from jax import lax
from jax.experimental import pallas as pl
from jax.experimental.pallas import tpu as pltpu
from jax.experimental.pallas import tpu_sc as plsc

G = 2
NUMQ = 2
K = 8192
D = 256
M = 2048
KBLK = 1024
NKB = K // KBLK


def _layer_step(xs_ref, xd_ref, cb_ref, kb, idx_ref, accv, acckb, xnb, rbuf):
    @pl.when(kb == 0)
    def _():
        if xd_ref is None:
            r = xs_ref[0]
        else:
            r = xs_ref[0] + 2.0 * xd_ref[0]
        rbuf[...] = r
        xn = jnp.sum(r * r, axis=1) * 0.25
        xnb[...] = jnp.broadcast_to(xn[:, None], (M, KBLK))

    cb = cb_ref[0, 0]
    cn = jnp.sum(cb * cb, axis=1)
    d = lax.dot_general(rbuf[...], cb, (((1,), (1,)), ((), ())),
                        preferred_element_type=jnp.float32)
    s = (xnb[...] + d) + cn[None, :]

    @pl.when(kb == 0)
    def _():
        accv[...] = s
        acckb[...] = jnp.zeros((M, KBLK), jnp.int8)

    @pl.when(kb > 0)
    def _():
        prev = accv[...]
        upd = s < prev
        accv[...] = jnp.minimum(s, prev)
        acckb[...] = jnp.where(upd, jnp.full((M, KBLK), kb, jnp.int8),
                               acckb[...])

    @pl.when(kb == NKB - 1)
    def _():
        av = accv[...]
        m = jnp.min(av, axis=1)
        jcol = lax.broadcasted_iota(jnp.int32, (M, KBLK), 1)
        gidx = acckb[...].astype(jnp.int32) * KBLK + jcol
        cand = jnp.where(av == m[:, None], gidx, K)
        idx_ref[0] = jnp.min(cand, axis=1)[None]


def _layer1_body(x_ref, cb_ref, idx_ref, accv, acckb, xnb, rbuf):
    _layer_step(x_ref, None, cb_ref, pl.program_id(1), idx_ref, accv, acckb,
                xnb, rbuf)


def _layer2_body(x_ref, xd_ref, cb_ref, idx_ref, accv, acckb, xnb, rbuf):
    _layer_step(x_ref, xd_ref, cb_ref, pl.program_id(1), idx_ref, accv,
                acckb, xnb, rbuf)


def _finalize_body(x_ref, xd0_ref, xd1_ref, cnt_ref, q_ref, cp_ref, perp_ref):
    xf = x_ref[0]
    a = xd0_ref[0]
    b = xd1_ref[0]
    q_ref[0] = a + b
    r0 = xf - a
    c0 = jnp.sum((r0 * r0).reshape(1, M * D), axis=1, keepdims=True)
    e = r0 - b
    c1 = jnp.sum((e * e).reshape(1, M * D), axis=1, keepdims=True)
    cp_ref[0] = (c0 + c1) * (1.0 / (M * D))
    prob = cnt_ref[0] * (1.0 / M)
    h = jnp.sum(prob * jnp.log(prob + 1e-07), axis=1, keepdims=True)
    perp_ref[0] = jnp.exp(-h)


_x_spec = pl.BlockSpec((1, M, D), lambda g, kb: (g, 0, 0))
def _cb_spec(q):
    return pl.BlockSpec((1, 1, KBLK, D), lambda g, kb: (g, q, kb, 0))
_idx_spec = pl.BlockSpec((1, 1, M), lambda g, kb: (g, 0, 0))


def _run_layer1(xf, cbs):
    return pl.pallas_call(
        _layer1_body,
        grid=(G, NKB),
        in_specs=[_x_spec, _cb_spec(0)],
        out_specs=[_idx_spec],
        out_shape=[jax.ShapeDtypeStruct((G, 1, M), jnp.int32)],
        scratch_shapes=[pltpu.VMEM((M, KBLK), jnp.float32),
                        pltpu.VMEM((M, KBLK), jnp.int8),
                        pltpu.VMEM((M, KBLK), jnp.float32),
                        pltpu.VMEM((M, D), jnp.float32)],
        compiler_params=pltpu.CompilerParams(
            vmem_limit_bytes=100 * 1024 * 1024),
    )(xf, cbs)[0]


def _run_layer2(xf, xd0, cbs):
    return pl.pallas_call(
        _layer2_body,
        grid=(G, NKB),
        in_specs=[_x_spec, _x_spec, _cb_spec(1)],
        out_specs=[_idx_spec],
        out_shape=[jax.ShapeDtypeStruct((G, 1, M), jnp.int32)],
        scratch_shapes=[pltpu.VMEM((M, KBLK), jnp.float32),
                        pltpu.VMEM((M, KBLK), jnp.int8),
                        pltpu.VMEM((M, KBLK), jnp.float32),
                        pltpu.VMEM((M, D), jnp.float32)],
        compiler_params=pltpu.CompilerParams(
            vmem_limit_bytes=100 * 1024 * 1024),
    )(xf, xd0, cbs)[0]


def _run_finalize(xf, xd0, xd1, counts):
    s_x = pl.BlockSpec((1, M, D), lambda g: (g, 0, 0))
    s_s = pl.BlockSpec((1, 1, 1), lambda g: (g, 0, 0))
    s_c = pl.BlockSpec((1, 1, K), lambda g: (g, 0, 0))
    return pl.pallas_call(
        _finalize_body,
        grid=(G,),
        in_specs=[s_x, s_x, s_x, s_c],
        out_specs=[s_x, s_s, s_s],
        out_shape=[jax.ShapeDtypeStruct((G, M, D), jnp.float32),
                   jax.ShapeDtypeStruct((G, 1, 1), jnp.float32),
                   jax.ShapeDtypeStruct((G, 1, 1), jnp.float32)],
    )(xf, xd0, xd1, counts)


_NC, _NS = 2, 16
_RPW = (G * M) // (_NC * _NS)
_CPW = K // _NS


def _sc_gather(cb_flat, gidx_flat, lidx_flat=None, ones=None, zeros=None):
    mesh = plsc.VectorSubcoreMesh(core_axis_name="c", subcore_axis_name="s")
    with_counts = lidx_flat is not None
    out_type = [jax.ShapeDtypeStruct((G * M, D), jnp.float32)]
    if with_counts:
        out_type.append(jax.ShapeDtypeStruct((G * K,), jnp.float32))
    scratch = [
        pltpu.VMEM((_RPW,), jnp.int32),
        pltpu.VMEM((_RPW, D), jnp.float32),
        pltpu.SemaphoreType.DMA,
    ]
    if with_counts:
        scratch += [pltpu.VMEM((_RPW,), jnp.int32),
                    pltpu.VMEM((_RPW,), jnp.float32),
                    pltpu.VMEM_SHARED((K,), jnp.float32)]

    @functools.partial(pl.kernel, mesh=mesh, out_type=out_type,
                       scratch_types=scratch)
    def k(cb_hbm, gidx_hbm, *rest):
        if with_counts:
            (lidx_hbm, ones_hbm, zeros_hbm, out_hbm, cnt_hbm,
             idx_v, rows_v, sem, lidx_v, ones_v, shared) = rest
        else:
            out_hbm, idx_v, rows_v, sem = rest
        c = lax.axis_index("c")
        s = lax.axis_index("s")
        base = c * M + s * _RPW
        pltpu.sync_copy(gidx_hbm.at[pl.ds(base, _RPW)], idx_v)

        if with_counts:
            pltpu.sync_copy(lidx_hbm.at[pl.ds(base, _RPW)], lidx_v)
            pltpu.sync_copy(ones_hbm.at[pl.ds(base, _RPW)], ones_v)
            pltpu.sync_copy(zeros_hbm.at[pl.ds(s * _CPW, _CPW)],
                            shared.at[pl.ds(s * _CPW, _CPW)])
            plsc.subcore_barrier()
            pltpu.sync_copy(ones_v, shared.at[lidx_v], add=True)
            plsc.subcore_barrier()
            pltpu.sync_copy(shared.at[pl.ds(s * _CPW, _CPW)],
                            cnt_hbm.at[pl.ds(c * K + s * _CPW, _CPW)])

        pltpu.async_copy(cb_hbm.at[idx_v], rows_v, sem).wait()
        pltpu.sync_copy(rows_v, out_hbm.at[pl.ds(base, _RPW)])

    if with_counts:
        return k(cb_flat, gidx_flat, lidx_flat, ones, zeros)
    return k(cb_flat, gidx_flat)


def kernel(x, codebooks):
    xt = jnp.transpose(x, (0, 2, 1)).reshape(M, G * D)
    xf = jnp.stack([xt[:, g * D:(g + 1) * D] for g in range(G)])
    cb_flat = codebooks.reshape(G * NUMQ * K, D)

    goff0 = (jnp.arange(G, dtype=jnp.int32) * NUMQ * K)[:, None]
    ones = jnp.ones((G * M,), jnp.float32)
    zeros = jnp.zeros((K,), jnp.float32)

    xs = -2.0 * xf
    idx0 = _run_layer1(xs, codebooks)
    xd0, counts = _sc_gather(cb_flat,
                             (idx0.reshape(G, M) + goff0).reshape(G * M),
                             idx0.reshape(G * M), ones, zeros)
    xd0 = xd0.reshape(G, M, D)
    idx1 = _run_layer2(xs, xd0, codebooks)
    xd1 = _sc_gather(cb_flat,
                     (idx1.reshape(G, M) + goff0 + K).reshape(G * M)
                     )[0].reshape(G, M, D)
    q, cp, perp = _run_finalize(xf, xd0, xd1, counts.reshape(G, 1, K))

    quantized = jnp.transpose(
        jnp.concatenate([q[g].reshape(4, 512, D) for g in range(G)], axis=2),
        (0, 2, 1))
    commit_total = cp[0, 0, 0] + cp[1, 0, 0]
    return quantized, commit_total, perp.reshape(G), idx0.reshape(G, M)


# speedup vs baseline: 1.6887x; 1.0174x over previous
import functools

import jax
import jax.numpy as jnp
from jax import lax
from jax.experimental import pallas as pl
from jax.experimental.pallas import tpu as pltpu
from jax.experimental.pallas import tpu_sc as plsc

G = 2
NUMQ = 2
K = 8192
D = 256
M = 2048
KBLK = 1024
NKB = K // KBLK


def _layer_step(xs_ref, xd_ref, cb_ref, kb, idx_ref, accv, acckb, xnb, rbuf):
    @pl.when(kb == 0)
    def _():
        if xd_ref is None:
            r = xs_ref[...]
        else:
            r = xs_ref[...] + 2.0 * xd_ref[0]
        rbuf[...] = r
        xn = jnp.sum(r * r, axis=1) * 0.25
        xnb[...] = jnp.broadcast_to(xn[:, None], (M, KBLK))

    cb = cb_ref[0, 0]
    cn = jnp.sum(cb * cb, axis=1)
    d = lax.dot_general(rbuf[...], cb, (((1,), (1,)), ((), ())),
                        preferred_element_type=jnp.float32)
    s = (xnb[...] + d) + cn[None, :]

    @pl.when(kb == 0)
    def _():
        accv[...] = s
        acckb[...] = jnp.zeros((M, KBLK), jnp.int8)

    @pl.when(kb > 0)
    def _():
        prev = accv[...]
        upd = s < prev
        accv[...] = jnp.minimum(s, prev)
        acckb[...] = jnp.where(upd, jnp.full((M, KBLK), kb, jnp.int8),
                               acckb[...])

    @pl.when(kb == NKB - 1)
    def _():
        av = accv[...]
        m = jnp.min(av, axis=1)
        jcol = lax.broadcasted_iota(jnp.int32, (M, KBLK), 1)
        gidx = acckb[...].astype(jnp.int32) * KBLK + jcol
        cand = jnp.where(av == m[:, None], gidx, K)
        idx_ref[0] = jnp.min(cand, axis=1)[None]


def _layer1_body(x_ref, cb_ref, idx_ref, accv, acckb, xnb, rbuf):
    _layer_step(x_ref, None, cb_ref, pl.program_id(1), idx_ref, accv, acckb,
                xnb, rbuf)


def _layer2_body(x_ref, xd_ref, cb_ref, idx_ref, accv, acckb, xnb, rbuf):
    _layer_step(x_ref, xd_ref, cb_ref, pl.program_id(1), idx_ref, accv,
                acckb, xnb, rbuf)


def _finalize_body(x_ref, xd0_ref, xd1_ref, cnt_ref, q_ref, cp_ref, perp_ref):
    xf = x_ref[...]
    a = xd0_ref[0]
    b = xd1_ref[0]
    q_ref[...] = a + b
    r0 = xf - a
    c0 = jnp.sum((r0 * r0).reshape(1, M * D), axis=1, keepdims=True)
    e = r0 - b
    c1 = jnp.sum((e * e).reshape(1, M * D), axis=1, keepdims=True)
    cp_ref[0] = (c0 + c1) * (1.0 / (M * D))
    prob = cnt_ref[0] * (1.0 / M)
    h = jnp.sum(prob * jnp.log(prob + 1e-07), axis=1, keepdims=True)
    perp_ref[0] = jnp.exp(-h)


_x_spec = pl.BlockSpec((1, M, D), lambda g, kb: (g, 0, 0))
_xcol_spec = pl.BlockSpec((M, D), lambda g, kb: (0, g))
def _cb_spec(q):
    return pl.BlockSpec((1, 1, KBLK, D), lambda g, kb: (g, q, kb, 0))
_idx_spec = pl.BlockSpec((1, 1, M), lambda g, kb: (g, 0, 0))


def _run_layer1(xf, cbs):
    return pl.pallas_call(
        _layer1_body,
        grid=(G, NKB),
        in_specs=[_xcol_spec, _cb_spec(0)],
        out_specs=[_idx_spec],
        out_shape=[jax.ShapeDtypeStruct((G, 1, M), jnp.int32)],
        scratch_shapes=[pltpu.VMEM((M, KBLK), jnp.float32),
                        pltpu.VMEM((M, KBLK), jnp.int8),
                        pltpu.VMEM((M, KBLK), jnp.float32),
                        pltpu.VMEM((M, D), jnp.float32)],
        compiler_params=pltpu.CompilerParams(
            vmem_limit_bytes=100 * 1024 * 1024),
    )(xf, cbs)[0]


def _run_layer2(xf, xd0, cbs):
    return pl.pallas_call(
        _layer2_body,
        grid=(G, NKB),
        in_specs=[_xcol_spec, _x_spec, _cb_spec(1)],
        out_specs=[_idx_spec],
        out_shape=[jax.ShapeDtypeStruct((G, 1, M), jnp.int32)],
        scratch_shapes=[pltpu.VMEM((M, KBLK), jnp.float32),
                        pltpu.VMEM((M, KBLK), jnp.int8),
                        pltpu.VMEM((M, KBLK), jnp.float32),
                        pltpu.VMEM((M, D), jnp.float32)],
        compiler_params=pltpu.CompilerParams(
            vmem_limit_bytes=100 * 1024 * 1024),
    )(xf, xd0, cbs)[0]


def _run_finalize(xt, xd0, xd1, counts):
    s_x = pl.BlockSpec((1, M, D), lambda g: (g, 0, 0))
    s_xc = pl.BlockSpec((M, D), lambda g: (0, g))
    s_s = pl.BlockSpec((1, 1, 1), lambda g: (g, 0, 0))
    s_c = pl.BlockSpec((1, 1, K), lambda g: (g, 0, 0))
    return pl.pallas_call(
        _finalize_body,
        grid=(G,),
        in_specs=[s_xc, s_x, s_x, s_c],
        out_specs=[s_xc, s_s, s_s],
        out_shape=[jax.ShapeDtypeStruct((M, G * D), jnp.float32),
                   jax.ShapeDtypeStruct((G, 1, 1), jnp.float32),
                   jax.ShapeDtypeStruct((G, 1, 1), jnp.float32)],
    )(xt, xd0, xd1, counts)


_NC, _NS = 2, 16
_RPW = (G * M) // (_NC * _NS)
_CPW = K // _NS


def _sc_gather(cb_flat, gidx_flat, lidx_flat=None, ones=None, zeros=None):
    mesh = plsc.VectorSubcoreMesh(core_axis_name="c", subcore_axis_name="s")
    with_counts = lidx_flat is not None
    out_type = [jax.ShapeDtypeStruct((G * M, D), jnp.float32)]
    if with_counts:
        out_type.append(jax.ShapeDtypeStruct((G * K,), jnp.float32))
    scratch = [
        pltpu.VMEM((_RPW,), jnp.int32),
        pltpu.VMEM((_RPW, D), jnp.float32),
        pltpu.SemaphoreType.DMA,
    ]
    if with_counts:
        scratch += [pltpu.VMEM((_RPW,), jnp.int32),
                    pltpu.VMEM((_RPW,), jnp.float32),
                    pltpu.VMEM_SHARED((K,), jnp.float32)]

    @functools.partial(pl.kernel, mesh=mesh, out_type=out_type,
                       scratch_types=scratch)
    def k(cb_hbm, gidx_hbm, *rest):
        if with_counts:
            (lidx_hbm, ones_hbm, zeros_hbm, out_hbm, cnt_hbm,
             idx_v, rows_v, sem, lidx_v, ones_v, shared) = rest
        else:
            out_hbm, idx_v, rows_v, sem = rest
        c = lax.axis_index("c")
        s = lax.axis_index("s")
        base = c * M + s * _RPW
        pltpu.sync_copy(gidx_hbm.at[pl.ds(base, _RPW)], idx_v)

        if with_counts:
            pltpu.sync_copy(lidx_hbm.at[pl.ds(base, _RPW)], lidx_v)
            pltpu.sync_copy(ones_hbm.at[pl.ds(base, _RPW)], ones_v)
            pltpu.sync_copy(zeros_hbm.at[pl.ds(s * _CPW, _CPW)],
                            shared.at[pl.ds(s * _CPW, _CPW)])
            plsc.subcore_barrier()
            pltpu.sync_copy(ones_v, shared.at[lidx_v], add=True)
            plsc.subcore_barrier()
            pltpu.sync_copy(shared.at[pl.ds(s * _CPW, _CPW)],
                            cnt_hbm.at[pl.ds(c * K + s * _CPW, _CPW)])

        pltpu.async_copy(cb_hbm.at[idx_v], rows_v, sem).wait()
        pltpu.sync_copy(rows_v, out_hbm.at[pl.ds(base, _RPW)])

    if with_counts:
        return k(cb_flat, gidx_flat, lidx_flat, ones, zeros)
    return k(cb_flat, gidx_flat)


def kernel(x, codebooks):
    xt = jnp.transpose(x, (0, 2, 1)).reshape(M, G * D)
    cb_flat = codebooks.reshape(G * NUMQ * K, D)

    goff0 = (jnp.arange(G, dtype=jnp.int32) * NUMQ * K)[:, None]
    ones = jnp.ones((G * M,), jnp.float32)
    zeros = jnp.zeros((K,), jnp.float32)

    xs = -2.0 * xt
    idx0 = _run_layer1(xs, codebooks)
    xd0, counts = _sc_gather(cb_flat,
                             (idx0.reshape(G, M) + goff0).reshape(G * M),
                             idx0.reshape(G * M), ones, zeros)
    xd0 = xd0.reshape(G, M, D)
    idx1 = _run_layer2(xs, xd0, codebooks)
    xd1 = _sc_gather(cb_flat,
                     (idx1.reshape(G, M) + goff0 + K).reshape(G * M)
                     )[0].reshape(G, M, D)
    q, cp, perp = _run_finalize(xt, xd0, xd1, counts.reshape(G, 1, K))

    quantized = jnp.transpose(q.reshape(4, 512, G * D), (0, 2, 1))
    commit_total = cp[0, 0, 0] + cp[1, 0, 0]
    return quantized, commit_total, perp.reshape(G), idx0.reshape(G, M)


# speedup vs baseline: 1.7207x; 1.0190x over previous
import functools

import jax
import jax.numpy as jnp
from jax import lax
from jax.experimental import pallas as pl
from jax.experimental.pallas import tpu as pltpu
from jax.experimental.pallas import tpu_sc as plsc

G = 2
NUMQ = 2
K = 8192
D = 256
M = 2048
KBLK = 1024
NKB = K // KBLK


def _layer_step(xs_ref, xd_ref, cb_ref, kb, idx_ref, accv, acckb, xnb, rbuf):
    @pl.when(kb == 0)
    def _():
        if xd_ref is None:
            r = xs_ref[...]
        else:
            r = xs_ref[...] + 2.0 * xd_ref[0]
        rbuf[...] = r
        xn = jnp.sum(r * r, axis=1) * 0.25
        xnb[...] = jnp.broadcast_to(xn[:, None], (M, KBLK))

    cb = cb_ref[0, 0]
    cn = jnp.sum(cb * cb, axis=1)
    d = lax.dot_general(rbuf[...], cb, (((1,), (1,)), ((), ())),
                        preferred_element_type=jnp.float32)
    s = (xnb[...] + d) + cn[None, :]

    @pl.when(kb == 0)
    def _():
        accv[...] = s
        acckb[...] = jnp.zeros((M, KBLK), jnp.int8)

    @pl.when(kb > 0)
    def _():
        prev = accv[...]
        upd = s < prev
        accv[...] = jnp.minimum(s, prev)
        acckb[...] = jnp.where(upd, jnp.full((M, KBLK), kb, jnp.int8),
                               acckb[...])

    @pl.when(kb == NKB - 1)
    def _():
        av = accv[...]
        m = jnp.min(av, axis=1)
        jcol = lax.broadcasted_iota(jnp.int32, (M, KBLK), 1)
        gidx = acckb[...].astype(jnp.int32) * KBLK + jcol
        cand = jnp.where(av == m[:, None], gidx, K)
        idx_ref[0] = jnp.min(cand, axis=1)[None]


def _layer1_body(x_ref, cb_ref, idx_ref, accv, acckb, xnb, rbuf):
    _layer_step(x_ref, None, cb_ref, pl.program_id(1), idx_ref, accv, acckb,
                xnb, rbuf)


def _layer2_body(x_ref, xd_ref, cb_ref, idx_ref, accv, acckb, xnb, rbuf):
    _layer_step(x_ref, xd_ref, cb_ref, pl.program_id(1), idx_ref, accv,
                acckb, xnb, rbuf)


def _finalize_body(x_ref, xd0_ref, xd1_ref, cnt_ref, q_ref, cp_ref, perp_ref):
    xf = x_ref[...]
    a = xd0_ref[0]
    b = xd1_ref[0]
    q_ref[...] = a + b
    r0 = xf - a
    c0 = jnp.sum((r0 * r0).reshape(1, M * D), axis=1, keepdims=True)
    e = r0 - b
    c1 = jnp.sum((e * e).reshape(1, M * D), axis=1, keepdims=True)
    cp_ref[0] = (c0 + c1) * (1.0 / (M * D))
    prob = cnt_ref[0] * (1.0 / M)
    h = jnp.sum(prob * jnp.log(prob + 1e-07), axis=1, keepdims=True)
    perp_ref[0] = jnp.exp(-h)


_x_spec = pl.BlockSpec((1, M, D), lambda g, kb: (g, 0, 0))
_xcol_spec = pl.BlockSpec((M, D), lambda g, kb: (0, g))
def _cb_spec(q):
    return pl.BlockSpec((1, 1, KBLK, D), lambda g, kb: (g, q, kb, 0))
_idx_spec = pl.BlockSpec((1, 1, M), lambda g, kb: (g, 0, 0))


def _run_layer1(xf, cbs):
    return pl.pallas_call(
        _layer1_body,
        grid=(G, NKB),
        in_specs=[_xcol_spec, _cb_spec(0)],
        out_specs=[_idx_spec],
        out_shape=[jax.ShapeDtypeStruct((G, 1, M), jnp.int32)],
        scratch_shapes=[pltpu.VMEM((M, KBLK), jnp.float32),
                        pltpu.VMEM((M, KBLK), jnp.int8),
                        pltpu.VMEM((M, KBLK), jnp.float32),
                        pltpu.VMEM((M, D), jnp.float32)],
        compiler_params=pltpu.CompilerParams(
            vmem_limit_bytes=100 * 1024 * 1024),
    )(xf, cbs)[0]


def _run_layer2(xf, xd0, cbs):
    return pl.pallas_call(
        _layer2_body,
        grid=(G, NKB),
        in_specs=[_xcol_spec, _x_spec, _cb_spec(1)],
        out_specs=[_idx_spec],
        out_shape=[jax.ShapeDtypeStruct((G, 1, M), jnp.int32)],
        scratch_shapes=[pltpu.VMEM((M, KBLK), jnp.float32),
                        pltpu.VMEM((M, KBLK), jnp.int8),
                        pltpu.VMEM((M, KBLK), jnp.float32),
                        pltpu.VMEM((M, D), jnp.float32)],
        compiler_params=pltpu.CompilerParams(
            vmem_limit_bytes=100 * 1024 * 1024),
    )(xf, xd0, cbs)[0]


def _run_finalize(xt, xd0, xd1, counts):
    s_x = pl.BlockSpec((1, M, D), lambda g: (g, 0, 0))
    s_xc = pl.BlockSpec((M, D), lambda g: (0, g))
    s_s = pl.BlockSpec((1, 1, 1), lambda g: (g, 0, 0))
    s_c = pl.BlockSpec((1, 1, K), lambda g: (g, 0, 0))
    return pl.pallas_call(
        _finalize_body,
        grid=(G,),
        in_specs=[s_xc, s_x, s_x, s_c],
        out_specs=[s_xc, s_s, s_s],
        out_shape=[jax.ShapeDtypeStruct((M, G * D), jnp.float32),
                   jax.ShapeDtypeStruct((G, 1, 1), jnp.float32),
                   jax.ShapeDtypeStruct((G, 1, 1), jnp.float32)],
    )(xt, xd0, xd1, counts)


_NC, _NS = 2, 16
_RPW = (G * M) // (_NC * _NS)
_CPW = K // _NS


def _sc_gather(cb_flat, gidx_flat):
    mesh = plsc.VectorSubcoreMesh(core_axis_name="c", subcore_axis_name="s")

    @functools.partial(
        pl.kernel, mesh=mesh,
        out_type=jax.ShapeDtypeStruct((G * M, D), jnp.float32),
        scratch_types=[pltpu.VMEM((_RPW,), jnp.int32),
                       pltpu.VMEM((_RPW, D), jnp.float32),
                       pltpu.SemaphoreType.DMA])
    def k(cb_hbm, gidx_hbm, out_hbm, idx_v, rows_v, sem):
        c = lax.axis_index("c")
        s = lax.axis_index("s")
        base = c * M + s * _RPW
        pltpu.sync_copy(gidx_hbm.at[pl.ds(base, _RPW)], idx_v)
        pltpu.async_copy(cb_hbm.at[idx_v], rows_v, sem).wait()
        pltpu.sync_copy(rows_v, out_hbm.at[pl.ds(base, _RPW)])

    return k(cb_flat, gidx_flat)


def _sc_counts(lidx_flat, ones, zeros):
    mesh = plsc.VectorSubcoreMesh(core_axis_name="c", subcore_axis_name="s")

    @functools.partial(
        pl.kernel, mesh=mesh,
        out_type=jax.ShapeDtypeStruct((G * K,), jnp.float32),
        scratch_types=[pltpu.VMEM((_RPW,), jnp.int32),
                       pltpu.VMEM((_RPW,), jnp.float32),
                       pltpu.VMEM_SHARED((K,), jnp.float32)])
    def k(lidx_hbm, ones_hbm, zeros_hbm, cnt_hbm, lidx_v, ones_v, shared):
        c = lax.axis_index("c")
        s = lax.axis_index("s")
        base = c * M + s * _RPW
        pltpu.sync_copy(lidx_hbm.at[pl.ds(base, _RPW)], lidx_v)
        pltpu.sync_copy(ones_hbm.at[pl.ds(base, _RPW)], ones_v)
        pltpu.sync_copy(zeros_hbm.at[pl.ds(s * _CPW, _CPW)],
                        shared.at[pl.ds(s * _CPW, _CPW)])
        plsc.subcore_barrier()
        pltpu.sync_copy(ones_v, shared.at[lidx_v], add=True)
        plsc.subcore_barrier()
        pltpu.sync_copy(shared.at[pl.ds(s * _CPW, _CPW)],
                        cnt_hbm.at[pl.ds(c * K + s * _CPW, _CPW)])

    return k(lidx_flat, ones, zeros)


def kernel(x, codebooks):
    xt = jnp.transpose(x, (0, 2, 1)).reshape(M, G * D)
    cb_flat = codebooks.reshape(G * NUMQ * K, D)

    goff0 = (jnp.arange(G, dtype=jnp.int32) * NUMQ * K)[:, None]
    ones = jnp.ones((G * M,), jnp.float32)
    zeros = jnp.zeros((K,), jnp.float32)

    xs = -2.0 * xt
    idx0 = _run_layer1(xs, codebooks)
    counts = _sc_counts(idx0.reshape(G * M), ones, zeros)
    xd0 = _sc_gather(cb_flat,
                     (idx0.reshape(G, M) + goff0).reshape(G * M)
                     ).reshape(G, M, D)
    idx1 = _run_layer2(xs, xd0, codebooks)
    xd1 = _sc_gather(cb_flat,
                     (idx1.reshape(G, M) + goff0 + K).reshape(G * M)
                     ).reshape(G, M, D)
    q, cp, perp = _run_finalize(xt, xd0, xd1, counts.reshape(G, 1, K))

    quantized = jnp.transpose(q.reshape(4, 512, G * D), (0, 2, 1))
    commit_total = cp[0, 0, 0] + cp[1, 0, 0]
    return quantized, commit_total, perp.reshape(G), idx0.reshape(G, M)
